# baseline (speedup 1.0000x reference)
import jax
import jax.numpy as jnp
from jax import lax
from jax.experimental import pallas as pl

FEAT_H = 16
FEAT_W = 16
HW = FEAT_H * FEAT_W
GAUSS_TH = 0.6
EXPAND = 1.0
SIGMA = 1.6
BIN_OFF = 0.5
RADIUS = 1.0
NEG_LOG_TH = 0.5108255840295616
IMGS = 2
N_PER_IMG = 512
G_PER_IMG = 64


def _label_kernel(boxes_ref, gt_ref, flag_ref, gid_ref,
                  cls_ref, clsw_ref, reg_ref, regw_ref):
    rows = IMGS * N_PER_IMG
    ng = IMGS * G_PER_IMG
    boxes = boxes_ref[...].reshape(rows, 4)
    gt = gt_ref[...].reshape(ng, 8)
    flag = flag_ref[...].reshape(rows, 1)
    gid = gid_ref[...].reshape(rows, 1)

    goff = (lax.broadcasted_iota(jnp.int32, (rows, 1), 0)
            // N_PER_IMG) * G_PER_IMG
    gslot = gid + goff
    onehot = (gslot == lax.broadcasted_iota(jnp.int32, (rows, ng), 1)
              ).astype(jnp.float32)
    matched = jnp.dot(onehot, gt, preferred_element_type=jnp.float32,
                      precision=lax.Precision.HIGHEST)

    x1 = boxes[:, 0:1]
    y1 = boxes[:, 1:2]
    x2 = boxes[:, 2:3]
    y2 = boxes[:, 3:4]
    cx = (x1 + x2) * 0.5
    cy = (y1 + y2) * 0.5
    w = (x2 - x1 + 1.0) * EXPAND
    h = (y2 - y1 + 1.0) * EXPAND
    bx1 = cx - (w - 1.0) * 0.5
    by1 = cy - (h - 1.0) * 0.5
    bx2 = cx + (w - 1.0) * 0.5
    by2 = cy + (h - 1.0) * 0.5

    kx = matched[:, 4:5]
    ky = matched[:, 5:6]
    kv = matched[:, 6:7]

    sx = FEAT_W / (bx2 - bx1 + 1.0)
    sy = FEAT_H / (by2 - by1 + 1.0)
    x0 = (kx - bx1) * sx
    y0 = (ky - by1) * sy

    col = lax.broadcasted_iota(jnp.int32, (rows, HW), 1)
    bin_x = (col % FEAT_W).astype(jnp.float32)
    bin_y = (col // FEAT_W).astype(jnp.float32)

    dx = bin_x + BIN_OFF - x0
    dy = bin_y + BIN_OFF - y0
    inv2s2 = 1.0 / (2.0 * SIGMA ** 2)
    arg = dx * dx * inv2s2 + dy * dy * inv2s2
    score = jnp.exp(-arg)
    keep = arg <= NEG_LOG_TH

    vis = kv != 0.0
    pos = flag > 0
    active = pos & vis & jnp.any(keep, axis=-1, keepdims=True)

    cls_ref[...] = jnp.where(active, score, -1.0).reshape(IMGS, N_PER_IMG, HW)
    clsw_ref[...] = (jnp.where(active, 1.0, 0.0) * jnp.ones_like(score)
                     ).reshape(IMGS, N_PER_IMG, HW)

    m = active & keep
    off_x = (x0 - bin_x) / RADIUS
    off_y = (y0 - bin_y) / RADIUS
    zeros = jnp.zeros_like(score)
    reg = jnp.concatenate([jnp.where(m, off_x, zeros),
                           jnp.where(m, off_y, zeros)], axis=1)
    reg_ref[...] = reg.reshape(IMGS, N_PER_IMG, 2 * HW)
    rw = jnp.where(m, 1.0, 0.0)
    regw_ref[...] = jnp.concatenate([rw, rw], axis=1
                                    ).reshape(IMGS, N_PER_IMG, 2 * HW)


def kernel(boxes, gt_boxes, match_pos_flag, match_gt_id):
    B, N = boxes.shape[:2]
    KPS = 1

    flag = match_pos_flag.astype(jnp.int32).reshape(B, N, 1)
    gid = match_gt_id.astype(jnp.int32).reshape(B, N, 1)

    grid = (B // IMGS,)
    out_shapes = (
        jax.ShapeDtypeStruct((B, N, HW), jnp.float32),
        jax.ShapeDtypeStruct((B, N, HW), jnp.float32),
        jax.ShapeDtypeStruct((B, N, 2 * HW), jnp.float32),
        jax.ShapeDtypeStruct((B, N, 2 * HW), jnp.float32),
    )
    in_specs = [
        pl.BlockSpec((IMGS, N, 4), lambda i: (i, 0, 0)),
        pl.BlockSpec((IMGS, 64, 8), lambda i: (i, 0, 0)),
        pl.BlockSpec((IMGS, N, 1), lambda i: (i, 0, 0)),
        pl.BlockSpec((IMGS, N, 1), lambda i: (i, 0, 0)),
    ]
    out_specs = (
        pl.BlockSpec((IMGS, N, HW), lambda i: (i, 0, 0)),
        pl.BlockSpec((IMGS, N, HW), lambda i: (i, 0, 0)),
        pl.BlockSpec((IMGS, N, 2 * HW), lambda i: (i, 0, 0)),
        pl.BlockSpec((IMGS, N, 2 * HW), lambda i: (i, 0, 0)),
    )
    cls, clsw, reg, regw = pl.pallas_call(
        _label_kernel,
        grid=grid,
        in_specs=in_specs,
        out_specs=out_specs,
        out_shape=out_shapes,
    )(boxes, gt_boxes, flag, gid)

    return (cls.reshape(B, N, KPS, FEAT_H, FEAT_W),
            clsw.reshape(B, N, KPS, FEAT_H, FEAT_W),
            reg.reshape(B, N, 2 * KPS, FEAT_H, FEAT_W),
            regw.reshape(B, N, 2 * KPS, FEAT_H, FEAT_W))
